# Initial kernel scaffold; baseline (speedup 1.0000x reference)
import jax
import jax.numpy as jnp
from jax.experimental import pallas as pl


def kernel(src_corr_indices, tgt_corr_indices, src_scores, tgt_scores):
    raise NotImplementedError("write your pallas kernel here")


# speedup vs baseline: 11.2417x; 11.2417x over previous
import functools

import jax
import jax.numpy as jnp
from jax import lax
from jax.experimental import pallas as pl
from jax.experimental.pallas import tpu as pltpu
from jax.experimental.pallas import tpu_sc as plsc

N_SIDE = 100000
N_CORR = 50000
NUM_CORES = 2
NUM_SUBCORES = 16

LP = 100352
TILE_LAB = LP // NUM_SUBCORES
NP = 50048
TILE_IDX = NP // NUM_SUBCORES
DUMP = 100000

WEIGHT = 1.0
EPS = 1e-8
N_TOTAL = float(2 * N_SIDE)


def _sc_scatter_body(corr_hbm, zeros_hbm, ones_hbm, out_hbm,
                     shared, idx_v, buf_v, ones_v):
    c = lax.axis_index("c")
    s = lax.axis_index("s")
    pltpu.sync_copy(zeros_hbm, buf_v)
    pltpu.sync_copy(buf_v, shared.at[pl.ds(s * TILE_LAB, TILE_LAB)])
    pltpu.sync_copy(corr_hbm.at[pl.ds(c * NP + s * TILE_IDX, TILE_IDX)], idx_v)
    pltpu.sync_copy(ones_hbm, ones_v)
    plsc.subcore_barrier()
    pltpu.sync_copy(ones_v, shared.at[idx_v])
    plsc.subcore_barrier()
    pltpu.sync_copy(shared.at[pl.ds(s * TILE_LAB, TILE_LAB)], buf_v)
    pltpu.sync_copy(buf_v, out_hbm.at[pl.ds(c * LP + s * TILE_LAB, TILE_LAB)])


_sc_scatter = pl.kernel(
    _sc_scatter_body,
    out_type=jax.ShapeDtypeStruct((NUM_CORES * LP,), jnp.float32),
    mesh=plsc.VectorSubcoreMesh(
        core_axis_name="c", subcore_axis_name="s",
        num_cores=NUM_CORES, num_subcores=NUM_SUBCORES),
    scratch_types=[
        pltpu.VMEM_SHARED((LP,), jnp.float32),
        pltpu.VMEM((TILE_IDX,), jnp.int32),
        pltpu.VMEM((TILE_LAB,), jnp.float32),
        pltpu.VMEM((TILE_IDX,), jnp.float32),
    ],
)


def _tc_loss_body(ss_ref, ts_ref, lab_ref, out_ref):
    def side_sums(scores, labels):
        sc = jnp.clip(scores, 1e-7, 1.0 - 1e-7)
        pos = labels
        a = jnp.sum(pos * (-jnp.log(sc)))
        b = jnp.sum((1.0 - pos) * (-jnp.log(1.0 - sc)))
        p = jnp.sum(pos)
        pred = (scores > 0.5).astype(jnp.float32)
        cnt = jnp.sum(pred)
        d = jnp.sum(pred * pos)
        return p, a, b, cnt, d

    p1, a1, b1, c1, d1 = side_sums(ss_ref[...], lab_ref[0, pl.ds(0, N_SIDE)])
    p2, a2, b2, c2, d2 = side_sums(ts_ref[...], lab_ref[1, pl.ds(0, N_SIDE)])
    p = p1 + p2
    a = a1 + a2
    b = b1 + b2
    cnt = c1 + c2
    d = d1 + d2
    w_neg = p / N_TOTAL
    w_pos = 1.0 - w_neg
    loss = (w_pos * a + w_neg * b) / N_TOTAL * WEIGHT
    precision = d / (cnt + EPS)
    recall = d / (p + EPS)
    out_ref[0] = loss
    out_ref[1] = precision
    out_ref[2] = recall


_tc_loss = pl.pallas_call(
    _tc_loss_body,
    out_shape=jax.ShapeDtypeStruct((3,), jnp.float32),
    in_specs=[
        pl.BlockSpec((N_SIDE,), lambda: (0,)),
        pl.BlockSpec((N_SIDE,), lambda: (0,)),
        pl.BlockSpec((NUM_CORES, LP), lambda: (0, 0)),
    ],
    out_specs=pl.BlockSpec(memory_space=pltpu.SMEM),
)


def kernel(src_corr_indices, tgt_corr_indices, src_scores, tgt_scores):
    pad = jnp.full((NP - N_CORR,), DUMP, dtype=jnp.int32)
    corr = jnp.concatenate([src_corr_indices, pad, tgt_corr_indices, pad])
    zeros = jnp.zeros((TILE_LAB,), jnp.float32)
    ones = jnp.ones((TILE_IDX,), jnp.float32)
    labels = _sc_scatter(corr, zeros, ones).reshape(NUM_CORES, LP)
    out = _tc_loss(src_scores, tgt_scores, labels)
    return out[0], out[1], out[2]
